# Initial kernel scaffold; baseline (speedup 1.0000x reference)
import jax
import jax.numpy as jnp
from jax.experimental import pallas as pl


def kernel(x, weight):
    raise NotImplementedError("write your pallas kernel here")


# speedup vs baseline: 1.7974x; 1.7974x over previous
import functools

import jax
import jax.numpy as jnp
from jax import lax
from jax.experimental import pallas as pl
from jax.experimental.pallas import tpu as pltpu
from jax.experimental.pallas import tpu_sc as plsc

_NC = 2
_NS = 16
_NW = _NC * _NS

_D = 64
_B = 16384 * 50
_BPW = _B // _NW
_CHUNK = 512
_NCHUNK = _BPW // _CHUNK


def _emb_body(idx_hbm, table_hbm, out_hbm, idx_v, rows_v, sem):
    wid = lax.axis_index("s") * _NC + lax.axis_index("c")
    base = wid * _BPW

    def chunk_body(i):
        off = base + i * _CHUNK
        pltpu.sync_copy(idx_hbm.at[pl.ds(off, _CHUNK)], idx_v)
        pltpu.async_copy(table_hbm.at[idx_v], rows_v, sem).wait()
        pltpu.sync_copy(rows_v, out_hbm.at[pl.ds(off, _CHUNK)])

    pl.loop(0, _NCHUNK)(chunk_body)


@jax.jit
def _emb(idx, weight):
    mesh = plsc.VectorSubcoreMesh(core_axis_name="c", subcore_axis_name="s")
    kern = functools.partial(
        pl.kernel,
        out_type=jax.ShapeDtypeStruct((_B, _D), jnp.float32),
        mesh=mesh,
        scratch_types=[
            pltpu.VMEM((_CHUNK,), jnp.int32),
            pltpu.VMEM((_CHUNK, _D), jnp.float32),
            pltpu.SemaphoreType.DMA,
        ],
        compiler_params=pltpu.CompilerParams(use_tc_tiling_on_sc=False),
    )(_emb_body)
    return kern(idx, weight)


def kernel(x, weight):
    idx = x.reshape(_B).astype(jnp.int32)
    out = _emb(idx, weight)
    return out.reshape(x.shape[0], x.shape[1], _D)


# speedup vs baseline: 1.8730x; 1.0421x over previous
import functools

import jax
import jax.numpy as jnp
from jax import lax
from jax.experimental import pallas as pl
from jax.experimental.pallas import tpu as pltpu
from jax.experimental.pallas import tpu_sc as plsc

_NC = 2
_NS = 16
_NW = _NC * _NS

_D = 64
_B = 16384 * 50
_BPW = _B // _NW
_CHUNK = 320
_NB = 4
_NCHUNK = _BPW // _CHUNK
_NG = _NCHUNK // _NB


def _emb_body(idx_hbm, table_hbm, out_hbm, idx_v, rows_v, *sems):
    sem_g = sems[:_NB]
    sem_o = sems[_NB:]
    wid = lax.axis_index("s") * _NC + lax.axis_index("c")
    base = wid * _BPW
    pltpu.sync_copy(idx_hbm.at[pl.ds(base, _BPW)], idx_v)

    def issue_gather(i, b):
        pltpu.async_copy(
            table_hbm.at[idx_v.at[pl.ds(i * _CHUNK, _CHUNK)]],
            rows_v.at[b], sem_g[b])

    def wait_gather(b):
        pltpu.make_async_copy(
            table_hbm.at[idx_v.at[pl.ds(0, _CHUNK)]],
            rows_v.at[b], sem_g[b]).wait()

    def wait_store(b):
        pltpu.make_async_copy(
            rows_v.at[b], out_hbm.at[pl.ds(base, _CHUNK)], sem_o[b]).wait()

    for b in range(_NB):
        issue_gather(b, b)

    def group(g):
        for b in range(_NB):
            i = g * _NB + b
            wait_gather(b)
            pltpu.async_copy(
                rows_v.at[b],
                out_hbm.at[pl.ds(base + i * _CHUNK, _CHUNK)], sem_o[b])

            @pl.when(g < _NG - 1)
            def _():
                wait_store(b)
                issue_gather(i + _NB, b)

    pl.loop(0, _NG)(group)
    for b in range(_NB):
        wait_store(b)


@jax.jit
def _emb(idx, weight):
    mesh = plsc.VectorSubcoreMesh(core_axis_name="c", subcore_axis_name="s")
    kern = functools.partial(
        pl.kernel,
        out_type=jax.ShapeDtypeStruct((_B, _D), jnp.float32),
        mesh=mesh,
        scratch_types=[
            pltpu.VMEM((_BPW,), jnp.int32),
            pltpu.VMEM((_NB, _CHUNK, _D), jnp.float32),
        ] + [pltpu.SemaphoreType.DMA] * (2 * _NB),
        compiler_params=pltpu.CompilerParams(use_tc_tiling_on_sc=False),
    )(_emb_body)
    return kern(idx, weight)


def kernel(x, weight):
    idx = x.reshape(_B).astype(jnp.int32)
    out = _emb(idx, weight)
    return out.reshape(x.shape[0], x.shape[1], _D)


# speedup vs baseline: 1.8758x; 1.0015x over previous
import functools

import jax
import jax.numpy as jnp
from jax import lax
from jax.experimental import pallas as pl
from jax.experimental.pallas import tpu as pltpu
from jax.experimental.pallas import tpu_sc as plsc

_NC = 2
_NS = 16
_NW = _NC * _NS

_D = 64
_B = 16384 * 50
_BPW = _B // _NW
_CHUNK = 320
_NB = 2
_NCHUNK = _BPW // _CHUNK
_NG = _NCHUNK // _NB


def _emb_body(idx_hbm, table_hbm, out_hbm, idx_v, rows_v, *sems):
    sem_g = sems[:_NB]
    sem_o = sems[_NB:]
    wid = lax.axis_index("s") * _NC + lax.axis_index("c")
    base = wid * _BPW
    pltpu.sync_copy(idx_hbm.at[pl.ds(base, _BPW)], idx_v)

    def issue_gather(i, b):
        pltpu.async_copy(
            table_hbm.at[idx_v.at[pl.ds(i * _CHUNK, _CHUNK)]],
            rows_v.at[b], sem_g[b])

    def wait_gather(b):
        pltpu.make_async_copy(
            table_hbm.at[idx_v.at[pl.ds(0, _CHUNK)]],
            rows_v.at[b], sem_g[b]).wait()

    def out_dst(i):
        return out_hbm.at[pl.ds(base + i * _CHUNK, _CHUNK)]

    def wait_store(b):
        pltpu.make_async_copy(rows_v.at[b], out_dst(0), sem_o[b]).wait()

    for b in range(_NB):
        issue_gather(b, b)

    def group(g):
        for b in range(_NB):
            i = g * _NB + b
            wait_gather(b)
            pltpu.async_copy(rows_v.at[b], out_dst(i), sem_o[b])

            @pl.when(g < _NG - 1)
            def _():
                wait_store(b)
                issue_gather(i + _NB, b)

    pl.loop(0, _NG)(group)
    for b in range(_NB):
        wait_store(b)


@jax.jit
def _emb(idx, table):
    mesh = plsc.VectorSubcoreMesh(core_axis_name="c", subcore_axis_name="s")
    kern = functools.partial(
        pl.kernel,
        out_type=jax.ShapeDtypeStruct((_B, _D), jnp.float32),
        mesh=mesh,
        scratch_types=[
            pltpu.VMEM((_BPW,), jnp.int32),
            pltpu.VMEM((_NB, _CHUNK, _D), jnp.float32),
        ] + [pltpu.SemaphoreType.DMA] * (2 * _NB),
        compiler_params=pltpu.CompilerParams(use_tc_tiling_on_sc=False),
    )(_emb_body)
    return kern(idx, table)


def kernel(x, weight):
    idx = x.reshape(_B).astype(jnp.int32)
    out = _emb(idx, weight)
    return out.reshape(x.shape[0], x.shape[1], _D)
